# baseline (speedup 1.0000x reference)
import functools
import math

import jax
import jax.numpy as jnp
from jax import lax
from jax.experimental import pallas as pl
from jax.experimental.pallas import tpu as pltpu
from jax.experimental.pallas import tpu_sc as plsc

_PAD = 0
_B, _N = 8, 256
_D = 512
_RBF = 64
_SK = 80
_L = 4
_CUTOFF = 6.0
_GAMMA = 10.0
_BN = _B * _N


def _sc_gather(embed, tokens_flat):
    info = plsc.get_sparse_core_info()
    nc, ns = info.num_cores, info.num_subcores
    nw = nc * ns
    rows_per_worker = _BN // nw

    mesh = plsc.VectorSubcoreMesh(core_axis_name="c", subcore_axis_name="s")

    @functools.partial(
        pl.kernel,
        mesh=mesh,
        out_type=jax.ShapeDtypeStruct((_BN, _D), jnp.float32),
        scratch_types=[
            pltpu.VMEM((rows_per_worker,), jnp.int32),
            pltpu.VMEM((rows_per_worker, _D), jnp.float32),
            pltpu.SemaphoreType.DMA,
        ],
    )
    def gather_k(table_hbm, idx_hbm, out_hbm, idx_v, rows_v, sem):
        wid = lax.axis_index("s") * nc + lax.axis_index("c")
        base = wid * rows_per_worker
        pltpu.sync_copy(idx_hbm.at[pl.ds(base, rows_per_worker)], idx_v)
        pltpu.async_copy(table_hbm.at[idx_v], rows_v, sem).wait()
        pltpu.sync_copy(rows_v, out_hbm.at[pl.ds(base, rows_per_worker)])

    return gather_k(embed, tokens_flat)


def _tc_body(crow_ref, ccol_ref, vrow_ref, vcol_ref, mcol_ref, x0_ref,
             g_ref, w1_ref, b1_ref, w2_ref, b2_ref, out_ref, s_ref, s3_ref,
             s3b_ref):
    s_ref[0:_SK, :] = jnp.zeros((_SK, _BN), jnp.float32)

    delta = _CUTOFF / (_RBF - 1)
    q0 = math.exp(-_GAMMA * delta * delta)
    qstep = q0 * q0
    half = _N // 2
    for b in range(_B):
        for jc in range(2):
            c0 = jc * half
            tcol = b * 2 + jc
            for ic in range(2):
                r0 = ic * half
                sdst = s3_ref if ic == 0 else s3b_ref
                dx = (ccol_ref[b, r0:r0 + half, 0:1]
                      - crow_ref[b, 0:1, c0:c0 + half])
                dy = (ccol_ref[b, r0:r0 + half, 1:2]
                      - crow_ref[b, 1:2, c0:c0 + half])
                dz = (ccol_ref[b, r0:r0 + half, 2:3]
                      - crow_ref[b, 2:3, c0:c0 + half])
                dist = jnp.sqrt(dx * dx + dy * dy + dz * dz)
                ri = r0 + lax.broadcasted_iota(jnp.int32, (half, half), 0)
                ci = c0 + lax.broadcasted_iota(jnp.int32, (half, half), 1)
                ok = ((dist < _CUTOFF) & (ri != ci)
                      & (vcol_ref[b, r0:r0 + half] > 0.5)
                      & (vrow_ref[b, 0:1, c0:c0 + half] > 0.5))
                adjf = jnp.where(ok, 1.0, 0.0).astype(jnp.float32)

                p = jnp.exp(jnp.minimum(2.0 * _GAMMA * delta * dist, 80.0))

                def rbf_row(k, carry):
                    e, q = carry
                    sdst[k, tcol, :] = jnp.sum(e, axis=0)
                    return e * (p * q), q * qstep

                chunk = 16
                for k0 in range(0, _RBF, chunk):
                    t = dist - (k0 * delta)
                    e_start = adjf * jnp.exp(-_GAMMA * t * t)
                    q_start = math.exp(-_GAMMA * delta * delta * (2 * k0 + 1))
                    lax.fori_loop(k0, k0 + chunk, rbf_row,
                                  (e_start, jnp.float32(q_start)))
                sdst[_RBF, tcol, :] = jnp.sum(adjf, axis=0)

    for t in range(2 * _B):
        s_ref[0:_RBF + 1, pl.ds(t * half, half)] = (
            s3_ref[0:_RBF + 1, t, :] + s3b_ref[0:_RBF + 1, t, :])

    st = s_ref[:, :].astype(jnp.bfloat16)
    acc = x0_ref[:, :]
    for l in range(_L):
        agg = lax.dot_general(
            st, g_ref[l],
            dimension_numbers=(((0,), (0,)), ((), ())),
            preferred_element_type=jnp.float32).astype(jnp.bfloat16)
        t1 = jnp.maximum(
            jnp.dot(agg, w1_ref[l], preferred_element_type=jnp.float32)
            + b1_ref[l], 0.0).astype(jnp.bfloat16)
        h = jnp.dot(t1, w2_ref[l], preferred_element_type=jnp.float32) + b2_ref[l]
        acc = acc + h
    out_ref[:, :] = acc * mcol_ref[:, :]


def _tc_chain(crow, ccol, vrow, vcol, mcol, x0, g, w1, b1, w2, b2):
    return pl.pallas_call(
        _tc_body,
        out_shape=jax.ShapeDtypeStruct((_BN, _D), jnp.float32),
        scratch_shapes=[pltpu.VMEM((_SK, _BN), jnp.float32),
                        pltpu.VMEM((_SK, 2 * _B, _N // 2), jnp.float32),
                        pltpu.VMEM((_SK, 2 * _B, _N // 2), jnp.float32)],
    )(crow, ccol, vrow, vcol, mcol, x0, g, w1, b1, w2, b2)


def kernel(src_tokens, padded_coordinates, src_distance, src_edge_type,
           embed, edge_W, edge_b, node_W1, node_b1, node_W2, node_b2):
    del src_distance, src_edge_type
    padding_mask = src_tokens == _PAD
    tokens_flat = src_tokens.reshape(_BN).astype(jnp.int32)
    x0 = _sc_gather(embed.astype(jnp.float32), tokens_flat)

    coords = padded_coordinates.astype(jnp.float32)
    crow = coords.transpose(0, 2, 1)
    ccol = coords
    validf = (~padding_mask).astype(jnp.float32)
    vrow = validf[:, None, :]
    vcol = validf[:, :, None]
    mcol = validf.reshape(_BN, 1)

    g = jnp.concatenate(
        [edge_W[:, :_RBF, :], edge_b[:, None, :],
         jnp.zeros((_L, _SK - _RBF - 1, _D), jnp.float32)],
        axis=1).astype(jnp.bfloat16)

    out = _tc_chain(crow, ccol, vrow, vcol, mcol, x0, g,
                    node_W1.astype(jnp.bfloat16), node_b1,
                    node_W2.astype(jnp.bfloat16), node_b2)
    return out.reshape(_B, _N, _D), padding_mask


# speedup vs baseline: 1.1562x; 1.1562x over previous
import functools
import math

import jax
import jax.numpy as jnp
from jax import lax
from jax.experimental import pallas as pl
from jax.experimental.pallas import tpu as pltpu
from jax.experimental.pallas import tpu_sc as plsc

_PAD = 0
_B, _N = 8, 256
_D = 512
_RBF = 64
_SK = 80
_L = 4
_CUTOFF = 6.0
_GAMMA = 10.0
_BN = _B * _N


def _sc_gather(embed, tokens_flat):
    info = plsc.get_sparse_core_info()
    nc, ns = info.num_cores, info.num_subcores
    nw = nc * ns
    rows_per_worker = _BN // nw

    mesh = plsc.VectorSubcoreMesh(core_axis_name="c", subcore_axis_name="s")

    @functools.partial(
        pl.kernel,
        mesh=mesh,
        out_type=jax.ShapeDtypeStruct((_BN, _D), jnp.float32),
        scratch_types=[
            pltpu.VMEM((rows_per_worker,), jnp.int32),
            pltpu.VMEM((rows_per_worker, _D), jnp.float32),
            pltpu.SemaphoreType.DMA,
        ],
    )
    def gather_k(table_hbm, idx_hbm, out_hbm, idx_v, rows_v, sem):
        wid = lax.axis_index("s") * nc + lax.axis_index("c")
        base = wid * rows_per_worker
        pltpu.sync_copy(idx_hbm.at[pl.ds(base, rows_per_worker)], idx_v)
        pltpu.async_copy(table_hbm.at[idx_v], rows_v, sem).wait()
        pltpu.sync_copy(rows_v, out_hbm.at[pl.ds(base, rows_per_worker)])

    return gather_k(embed, tokens_flat)


def _tc_body(crow_ref, ccol_ref, vrow_ref, vcol_ref, mcol_ref,
             g_ref, w1_ref, b1_ref, w2_ref, b2_ref, out_ref, s_ref):
    s_ref[_RBF + 1:_SK, :] = jnp.zeros((_SK - _RBF - 1, _BN), jnp.float32)

    delta = _CUTOFF / (_RBF - 1)
    q0 = math.exp(-_GAMMA * delta * delta)
    qstep = q0 * q0
    for b in range(_B):
        dx = ccol_ref[b, :, 0:1] - crow_ref[b, 0:1, :]
        dy = ccol_ref[b, :, 1:2] - crow_ref[b, 1:2, :]
        dz = ccol_ref[b, :, 2:3] - crow_ref[b, 2:3, :]
        dist = jnp.sqrt(dx * dx + dy * dy + dz * dz)
        ri = lax.broadcasted_iota(jnp.int32, (_N, _N), 0)
        ci = lax.broadcasted_iota(jnp.int32, (_N, _N), 1)
        ok = ((dist < _CUTOFF) & (ri != ci)
              & (vcol_ref[b] > 0.5) & (vrow_ref[b] > 0.5))
        adjf = jnp.where(ok, 1.0, 0.0).astype(jnp.float32)

        p = jnp.exp(jnp.minimum(2.0 * _GAMMA * delta * dist, 80.0))

        def rbf_row(k, carry):
            e, q = carry
            s_ref[k, pl.ds(b * _N, _N)] = jnp.sum(e, axis=0)
            return e * (p * q), q * qstep

        chunk = 16
        for k0 in range(0, _RBF, chunk):
            t = dist - (k0 * delta)
            e_start = adjf * jnp.exp(-_GAMMA * t * t)
            q_start = math.exp(-_GAMMA * delta * delta * (2 * k0 + 1))
            lax.fori_loop(k0, k0 + chunk, rbf_row,
                          (e_start, jnp.float32(q_start)))
        s_ref[_RBF, pl.ds(b * _N, _N)] = jnp.sum(adjf, axis=0)

    st = s_ref[:, :].astype(jnp.bfloat16)
    acc = jnp.zeros((_BN, _D), jnp.float32)
    for l in range(_L):
        agg = lax.dot_general(
            st, g_ref[l],
            dimension_numbers=(((0,), (0,)), ((), ())),
            preferred_element_type=jnp.float32).astype(jnp.bfloat16)
        t1 = jnp.maximum(
            jnp.dot(agg, w1_ref[l], preferred_element_type=jnp.float32)
            + b1_ref[l], 0.0).astype(jnp.bfloat16)
        h = jnp.dot(t1, w2_ref[l], preferred_element_type=jnp.float32) + b2_ref[l]
        acc = acc + h
    out_ref[:, :] = acc * mcol_ref[:, :]


def _tc_chain(crow, ccol, vrow, vcol, mcol, g, w1, b1, w2, b2):
    return pl.pallas_call(
        _tc_body,
        out_shape=jax.ShapeDtypeStruct((_BN, _D), jnp.float32),
        scratch_shapes=[pltpu.VMEM((_SK, _BN), jnp.float32)],
    )(crow, ccol, vrow, vcol, mcol, g, w1, b1, w2, b2)


def kernel(src_tokens, padded_coordinates, src_distance, src_edge_type,
           embed, edge_W, edge_b, node_W1, node_b1, node_W2, node_b2):
    del src_distance, src_edge_type
    padding_mask = src_tokens == _PAD
    tokens_flat = src_tokens.reshape(_BN).astype(jnp.int32)
    x0 = _sc_gather(embed.astype(jnp.float32), tokens_flat)

    coords = padded_coordinates.astype(jnp.float32)
    crow = coords.transpose(0, 2, 1)
    ccol = coords
    validf = (~padding_mask).astype(jnp.float32)
    vrow = validf[:, None, :]
    vcol = validf[:, :, None]
    mcol = validf.reshape(_BN, 1)

    g = jnp.concatenate(
        [edge_W[:, :_RBF, :], edge_b[:, None, :],
         jnp.zeros((_L, _SK - _RBF - 1, _D), jnp.float32)],
        axis=1).astype(jnp.bfloat16)

    h_sum = _tc_chain(crow, ccol, vrow, vcol, mcol, g,
                      node_W1.astype(jnp.bfloat16), node_b1,
                      node_W2.astype(jnp.bfloat16), node_b2)
    out = h_sum + x0
    return out.reshape(_B, _N, _D), padding_mask


# speedup vs baseline: 1.4115x; 1.2208x over previous
import functools
import math

import jax
import jax.numpy as jnp
from jax import lax
from jax.experimental import pallas as pl
from jax.experimental.pallas import tpu as pltpu
from jax.experimental.pallas import tpu_sc as plsc

_PAD = 0
_B, _N = 8, 256
_D = 512
_RBF = 64
_SK = 80
_L = 4
_CUTOFF = 6.0
_GAMMA = 10.0
_BN = _B * _N


def _sc_gather(embed, tokens_flat):
    info = plsc.get_sparse_core_info()
    nc, ns = info.num_cores, info.num_subcores
    nw = nc * ns
    rows_per_worker = _BN // nw

    mesh = plsc.VectorSubcoreMesh(core_axis_name="c", subcore_axis_name="s")

    @functools.partial(
        pl.kernel,
        mesh=mesh,
        out_type=jax.ShapeDtypeStruct((_BN, _D), jnp.float32),
        scratch_types=[
            pltpu.VMEM((rows_per_worker,), jnp.int32),
            pltpu.VMEM((rows_per_worker, _D), jnp.float32),
            pltpu.SemaphoreType.DMA,
        ],
    )
    def gather_k(table_hbm, idx_hbm, out_hbm, idx_v, rows_v, sem):
        wid = lax.axis_index("s") * nc + lax.axis_index("c")
        base = wid * rows_per_worker
        pltpu.sync_copy(idx_hbm.at[pl.ds(base, rows_per_worker)], idx_v)
        pltpu.async_copy(table_hbm.at[idx_v], rows_v, sem).wait()
        pltpu.sync_copy(rows_v, out_hbm.at[pl.ds(base, rows_per_worker)])

    return gather_k(embed, tokens_flat)


def _tc_body(crow_ref, ccol_ref, vrow_ref, vcol_ref, mcol_ref, x0_ref,
             g_ref, w1_ref, b1_ref, w2_ref, b2_ref, out_ref, sa_ref, sb_ref):
    sa_ref[_RBF + 1:_SK, :] = jnp.zeros((_SK - _RBF - 1, _BN), jnp.float32)
    sb_ref[_RBF + 1:_SK, :] = jnp.zeros((_SK - _RBF - 1, _BN), jnp.float32)

    delta = _CUTOFF / (_RBF - 1)
    w = math.exp(-_GAMMA * delta * delta)
    lnw = -_GAMMA * delta * delta
    half = _N // 2
    for b in range(_B):
        for jc in range(2):
            c0l = jc * half
            col = b * _N + c0l
            for ic in range(2):
                r0 = ic * half
                sdst = sa_ref if ic == 0 else sb_ref
                dx = (ccol_ref[b, r0:r0 + half, 0:1]
                      - crow_ref[b, 0:1, c0l:c0l + half])
                dy = (ccol_ref[b, r0:r0 + half, 1:2]
                      - crow_ref[b, 1:2, c0l:c0l + half])
                dz = (ccol_ref[b, r0:r0 + half, 2:3]
                      - crow_ref[b, 2:3, c0l:c0l + half])
                dist = jnp.sqrt(dx * dx + dy * dy + dz * dz)
                ri = r0 + lax.broadcasted_iota(jnp.int32, (half, half), 0)
                ci = c0l + lax.broadcasted_iota(jnp.int32, (half, half), 1)
                ok = ((dist < _CUTOFF) & (ri != ci)
                      & (vcol_ref[b, r0:r0 + half] > 0.5)
                      & (vrow_ref[b, 0:1, c0l:c0l + half] > 0.5))
                adjf = jnp.where(ok, 1.0, 0.0).astype(jnp.float32)
                p = jnp.exp(jnp.minimum(2.0 * _GAMMA * delta * dist, 80.0))

                def supergroup(sg, _, dist=dist, adjf=adjf, p=p, col=col,
                               sdst=sdst):
                    sgf = sg.astype(jnp.float32)
                    c0 = sgf * (16.0 * delta)
                    t = dist - c0
                    e = adjf * jnp.exp(-_GAMMA * t * t)
                    qsg = jnp.exp(jnp.full((1, half),
                                           lnw * (32.0 * sgf + 1.0),
                                           jnp.float32))
                    pq = p * qsg
                    rows = []
                    for j in range(16):
                        rows.append(jnp.sum(e, axis=0, keepdims=True))
                        if j < 15:
                            e = (e * (w ** (2 * j))) * pq
                    k0 = pl.multiple_of(sg * 16, 8)
                    k1 = pl.multiple_of(sg * 16 + 8, 8)
                    sdst[pl.ds(k0, 8), col:col + half] = (
                        jnp.concatenate(rows[0:8], axis=0))
                    sdst[pl.ds(k1, 8), col:col + half] = (
                        jnp.concatenate(rows[8:16], axis=0))
                    return 0

                lax.fori_loop(0, _RBF // 16, supergroup, 0)
                sdst[_RBF:_RBF + 1, col:col + half] = jnp.sum(
                    adjf, axis=0, keepdims=True)

    st = (sa_ref[:, :] + sb_ref[:, :]).astype(jnp.bfloat16)
    acc = x0_ref[:, :]
    for l in range(_L):
        agg = lax.dot_general(
            st, g_ref[l],
            dimension_numbers=(((0,), (0,)), ((), ())),
            preferred_element_type=jnp.float32).astype(jnp.bfloat16)
        t1 = jnp.maximum(
            jnp.dot(agg, w1_ref[l], preferred_element_type=jnp.float32)
            + b1_ref[l], 0.0).astype(jnp.bfloat16)
        h = jnp.dot(t1, w2_ref[l], preferred_element_type=jnp.float32) + b2_ref[l]
        acc = acc + h
    out_ref[:, :] = acc * mcol_ref[:, :]


def _tc_chain(crow, ccol, vrow, vcol, mcol, x0, g, w1, b1, w2, b2):
    return pl.pallas_call(
        _tc_body,
        out_shape=jax.ShapeDtypeStruct((_BN, _D), jnp.float32),
        scratch_shapes=[pltpu.VMEM((_SK, _BN), jnp.float32),
                        pltpu.VMEM((_SK, _BN), jnp.float32)],
    )(crow, ccol, vrow, vcol, mcol, x0, g, w1, b1, w2, b2)


def kernel(src_tokens, padded_coordinates, src_distance, src_edge_type,
           embed, edge_W, edge_b, node_W1, node_b1, node_W2, node_b2):
    del src_distance, src_edge_type
    padding_mask = src_tokens == _PAD
    tokens_flat = src_tokens.reshape(_BN).astype(jnp.int32)
    x0 = _sc_gather(embed.astype(jnp.float32), tokens_flat)

    coords = padded_coordinates.astype(jnp.float32)
    crow = coords.transpose(0, 2, 1)
    ccol = coords
    validf = (~padding_mask).astype(jnp.float32)
    vrow = validf[:, None, :]
    vcol = validf[:, :, None]
    mcol = validf.reshape(_BN, 1)

    g = jnp.concatenate(
        [edge_W[:, :_RBF, :], edge_b[:, None, :],
         jnp.zeros((_L, _SK - _RBF - 1, _D), jnp.float32)],
        axis=1).astype(jnp.bfloat16)

    out = _tc_chain(crow, ccol, vrow, vcol, mcol, x0, g,
                    node_W1.astype(jnp.bfloat16), node_b1,
                    node_W2.astype(jnp.bfloat16), node_b2)
    return out.reshape(_B, _N, _D), padding_mask


# speedup vs baseline: 1.4930x; 1.0577x over previous
import functools
import math

import jax
import jax.numpy as jnp
from jax import lax
from jax.experimental import pallas as pl
from jax.experimental.pallas import tpu as pltpu
from jax.experimental.pallas import tpu_sc as plsc

_PAD = 0
_B, _N = 8, 256
_D = 512
_RBF = 64
_SK = 80
_L = 4
_CUTOFF = 6.0
_GAMMA = 10.0
_BN = _B * _N


def _sc_gather(embed, tokens_flat):
    info = plsc.get_sparse_core_info()
    nc, ns = info.num_cores, info.num_subcores
    nw = nc * ns
    rows_per_worker = _BN // nw

    mesh = plsc.VectorSubcoreMesh(core_axis_name="c", subcore_axis_name="s")

    @functools.partial(
        pl.kernel,
        mesh=mesh,
        out_type=jax.ShapeDtypeStruct((_BN, _D), jnp.float32),
        scratch_types=[
            pltpu.VMEM((rows_per_worker,), jnp.int32),
            pltpu.VMEM((rows_per_worker, _D), jnp.float32),
            pltpu.SemaphoreType.DMA,
        ],
    )
    def gather_k(table_hbm, idx_hbm, out_hbm, idx_v, rows_v, sem):
        wid = lax.axis_index("s") * nc + lax.axis_index("c")
        base = wid * rows_per_worker
        pltpu.sync_copy(idx_hbm.at[pl.ds(base, rows_per_worker)], idx_v)
        pltpu.async_copy(table_hbm.at[idx_v], rows_v, sem).wait()
        pltpu.sync_copy(rows_v, out_hbm.at[pl.ds(base, rows_per_worker)])

    return gather_k(embed, tokens_flat)


def _tc_body(crow_ref, ccol_ref, vrow_ref, vcol_ref, mcol_ref, x0_ref,
             g_ref, w1_ref, b1_ref, w2_ref, b2_ref, out_ref, sa_ref, sb_ref):
    sa_ref[_RBF + 1:_RBF + 2, :] = jnp.ones((1, _BN), jnp.float32)
    sa_ref[_RBF + 2:_SK, :] = jnp.zeros((_SK - _RBF - 2, _BN), jnp.float32)
    sb_ref[_RBF + 1:_SK, :] = jnp.zeros((_SK - _RBF - 1, _BN), jnp.float32)

    delta = _CUTOFF / (_RBF - 1)
    w = math.exp(-_GAMMA * delta * delta)
    lnw = -_GAMMA * delta * delta
    half = _N // 2
    for b in range(_B):
        for jc in range(2):
            c0l = jc * half
            col = b * _N + c0l
            for ic in range(2):
                r0 = ic * half
                sdst = sa_ref if ic == 0 else sb_ref
                dx = (ccol_ref[b, r0:r0 + half, 0:1]
                      - crow_ref[b, 0:1, c0l:c0l + half])
                dy = (ccol_ref[b, r0:r0 + half, 1:2]
                      - crow_ref[b, 1:2, c0l:c0l + half])
                dz = (ccol_ref[b, r0:r0 + half, 2:3]
                      - crow_ref[b, 2:3, c0l:c0l + half])
                dist = jnp.sqrt(dx * dx + dy * dy + dz * dz)
                ri = r0 + lax.broadcasted_iota(jnp.int32, (half, half), 0)
                ci = c0l + lax.broadcasted_iota(jnp.int32, (half, half), 1)
                ok = ((dist < _CUTOFF) & (ri != ci)
                      & (vcol_ref[b, r0:r0 + half] > 0.5)
                      & (vrow_ref[b, 0:1, c0l:c0l + half] > 0.5))
                adjf = jnp.where(ok, 1.0, 0.0).astype(jnp.float32)
                p = jnp.exp(jnp.minimum(2.0 * _GAMMA * delta * dist, 80.0))

                def supergroup(sg, _, dist=dist, adjf=adjf, p=p, col=col,
                               sdst=sdst):
                    sgf = sg.astype(jnp.float32)
                    c0 = sgf * (16.0 * delta)
                    t = dist - c0
                    e = adjf * jnp.exp(-_GAMMA * t * t)
                    qsg = jnp.exp(jnp.full((1, half),
                                           lnw * (32.0 * sgf + 1.0),
                                           jnp.float32))
                    pq = p * qsg
                    rows = []
                    for j in range(16):
                        rows.append(jnp.sum(e, axis=0, keepdims=True))
                        if j < 15:
                            e = (e * (w ** (2 * j))) * pq
                    k0 = pl.multiple_of(sg * 16, 8)
                    k1 = pl.multiple_of(sg * 16 + 8, 8)
                    sdst[pl.ds(k0, 8), col:col + half] = (
                        jnp.concatenate(rows[0:8], axis=0))
                    sdst[pl.ds(k1, 8), col:col + half] = (
                        jnp.concatenate(rows[8:16], axis=0))
                    return 0

                lax.fori_loop(0, _RBF // 16, supergroup, 0)
                sdst[_RBF:_RBF + 1, col:col + half] = jnp.sum(
                    adjf, axis=0, keepdims=True)

    st = (sa_ref[:, :] + sb_ref[:, :]).astype(jnp.bfloat16)
    acc = x0_ref[:, :]
    rowid = lax.broadcasted_iota(jnp.int32, (_SK, 1), 0)
    for l in range(_L):
        gw1 = jnp.dot(g_ref[l], w1_ref[l], preferred_element_type=jnp.float32)
        gw1 = jnp.where(rowid == _RBF + 1, b1_ref[l][None, :], gw1)
        t1 = jnp.maximum(
            lax.dot_general(st, gw1.astype(jnp.bfloat16),
                            dimension_numbers=(((0,), (0,)), ((), ())),
                            preferred_element_type=jnp.float32),
            0.0).astype(jnp.bfloat16)
        h = jnp.dot(t1, w2_ref[l], preferred_element_type=jnp.float32) + b2_ref[l]
        acc = acc + h
    out_ref[:, :] = acc * mcol_ref[:, :]


def _tc_chain(crow, ccol, vrow, vcol, mcol, x0, g, w1, b1, w2, b2):
    return pl.pallas_call(
        _tc_body,
        out_shape=jax.ShapeDtypeStruct((_BN, _D), jnp.float32),
        scratch_shapes=[pltpu.VMEM((_SK, _BN), jnp.float32),
                        pltpu.VMEM((_SK, _BN), jnp.float32)],
    )(crow, ccol, vrow, vcol, mcol, x0, g, w1, b1, w2, b2)


def kernel(src_tokens, padded_coordinates, src_distance, src_edge_type,
           embed, edge_W, edge_b, node_W1, node_b1, node_W2, node_b2):
    del src_distance, src_edge_type
    padding_mask = src_tokens == _PAD
    tokens_flat = src_tokens.reshape(_BN).astype(jnp.int32)
    x0 = _sc_gather(embed.astype(jnp.float32), tokens_flat)

    coords = padded_coordinates.astype(jnp.float32)
    crow = coords.transpose(0, 2, 1)
    ccol = coords
    validf = (~padding_mask).astype(jnp.float32)
    vrow = validf[:, None, :]
    vcol = validf[:, :, None]
    mcol = validf.reshape(_BN, 1)

    g = jnp.concatenate(
        [edge_W[:, :_RBF, :], edge_b[:, None, :],
         jnp.zeros((_L, _SK - _RBF - 1, _D), jnp.float32)],
        axis=1).astype(jnp.bfloat16)

    out = _tc_chain(crow, ccol, vrow, vcol, mcol, x0, g,
                    node_W1.astype(jnp.bfloat16), node_b1,
                    node_W2.astype(jnp.bfloat16), node_b2)
    return out.reshape(_B, _N, _D), padding_mask
